# baseline (speedup 1.0000x reference)
import functools
import math

import jax
import jax.numpy as jnp
from jax import lax
from jax.experimental import pallas as pl
from jax.experimental.pallas import tpu as pltpu
from jax.experimental.pallas import tpu_sc as plsc

DIM = 128
HALF = 64
RANGE_MAX = 250.0
LOG_THETA = math.log(10000.0)

_NC = 2
_NS = 16
_NW = _NC * _NS
_K = 128


def _sc_gather(table, idx):
    B = idx.shape[0]
    b_per_w = B // _NW
    n_chunks = b_per_w // _K
    idx3 = idx.reshape(_NW, n_chunks, _K)
    mesh = plsc.VectorSubcoreMesh(core_axis_name="c", subcore_axis_name="s")

    @functools.partial(
        pl.kernel,
        mesh=mesh,
        out_type=jax.ShapeDtypeStruct((B, DIM), jnp.float32),
        scratch_types=[
            pltpu.VMEM((n_chunks, _K), jnp.int32),
            pltpu.VMEM((b_per_w, DIM), jnp.float32),
            pltpu.SemaphoreType.DMA,
        ],
    )
    def gather_kernel(table_hbm, idx_hbm, out_hbm, idx_v, rows_v, sem):
        wid = lax.axis_index("s") * _NC + lax.axis_index("c")
        pltpu.sync_copy(idx_hbm.at[wid], idx_v)
        copies = [
            pltpu.async_copy(
                table_hbm.at[idx_v.at[j]], rows_v.at[pl.ds(j * _K, _K)], sem
            )
            for j in range(n_chunks)
        ]
        for c in copies:
            c.wait()
        pltpu.sync_copy(rows_v, out_hbm.at[pl.ds(wid * b_per_w, b_per_w)])

    return gather_kernel(table, idx3)


def _sin01(x):
    x2 = x * x
    return x * (1.0 + x2 * (-1.0 / 6.0 + x2 * (1.0 / 120.0 + x2 * (-1.0 / 5040.0))))


def _cos01(x):
    x2 = x * x
    return 1.0 + x2 * (-0.5 + x2 * (1.0 / 24.0 + x2 * (-1.0 / 720.0 + x2 * (1.0 / 40320.0))))


def _make_dense_body(bs, grid, base, aliased):
    def body(*refs):
        if aliased:
            refs = refs[1:]
        (rows_ref, xt_ref, wd1, bd1, wd2, bd2, wc1, bc1, wc2, bc2,
         sep_ref, g_ref, b_ref, out_ref, sep_buf, d_buf, c_buf, sems) = refs
        i = pl.program_id(0)
        slot = lax.rem(i, 2)
        g = g_ref[...].reshape(1, DIM)
        b = b_ref[...].reshape(1, DIM)

        def ln(x):
            m = jnp.mean(x, axis=-1, keepdims=True)
            c = x - m
            v = jnp.mean(c * c, axis=-1, keepdims=True)
            return c * lax.rsqrt(v + 1e-5) * g + b

        def plane_dma(buf, step, plane, sem):
            return pltpu.make_async_copy(
                buf, out_ref.at[pl.ds(base + step * bs, bs), plane, :], sem)

        def wait_step(step, s):
            plane_dma(sep_buf, step, 0, sems.at[0, s]).wait()
            plane_dma(d_buf.at[s], step, 1, sems.at[1, s]).wait()
            plane_dma(c_buf.at[s], step, 2, sems.at[2, s]).wait()

        @pl.when(i >= 2)
        def _drain_old():
            wait_step(i - 2, slot)

        @pl.when(i == 0)
        def _fill_sep():
            sep_n = ln(sep_ref[...].reshape(1, DIM))
            sep_buf[...] = jnp.broadcast_to(sep_n, (bs, DIM))

        rows = rows_ref[...]
        h = jnp.dot(rows, wd1[...], preferred_element_type=jnp.float32)
        h = h + bd1[...].reshape(1, DIM)
        h = h * jax.nn.sigmoid(h)
        emb_d = jnp.dot(h, wd2[...], preferred_element_type=jnp.float32)
        emb_d = emb_d + bd2[...].reshape(1, DIM)
        d_buf[slot] = ln(emb_d)

        xc = jnp.clip(xt_ref[...].reshape(1, bs), 0.0, RANGE_MAX) * (1.0 / RANGE_MAX)
        f = lax.broadcasted_iota(jnp.int32, (HALF, 1), 0).astype(jnp.float32) * (1.0 / HALF)
        inv = jnp.exp(-LOG_THETA * f)
        ang = inv * xc
        feat_t = jnp.concatenate([_sin01(ang), _cos01(ang)], axis=0)
        h2 = lax.dot_general(feat_t, wc1[...], (((0,), (0,)), ((), ())),
                             preferred_element_type=jnp.float32)
        h2 = h2 + bc1[...].reshape(1, DIM)
        h2 = h2 * jax.nn.sigmoid(h2)
        emb_c = jnp.dot(h2, wc2[...], preferred_element_type=jnp.float32)
        emb_c = emb_c + bc2[...].reshape(1, DIM)
        c_buf[slot] = ln(emb_c)

        plane_dma(sep_buf, i, 0, sems.at[0, slot]).start()
        plane_dma(d_buf.at[slot], i, 1, sems.at[1, slot]).start()
        plane_dma(c_buf.at[slot], i, 2, sems.at[2, slot]).start()

        @pl.when(i == grid - 1)
        def _drain_tail():
            wait_step(i - 1, 1 - slot)
            wait_step(i, slot)

    return body


def _tc_dense_half(B_total, rows, xt, Wd1, bd1, Wd2, bd2, Wc1, bc1, Wc2, bc2,
                   sep, ln_g, ln_b, base, prev=None, interpret=False):
    Bh = rows.shape[0]
    bs = 4096
    grid = Bh // bs
    xt2 = xt.reshape(grid, 1, bs)
    w_spec = pl.BlockSpec((DIM, DIM), lambda i: (0, 0))
    b_spec = pl.BlockSpec((DIM,), lambda i: (0,))
    in_specs = [
        pl.BlockSpec((bs, DIM), lambda i: (i, 0)),
        pl.BlockSpec((1, 1, bs), lambda i: (i, 0, 0)),
        w_spec, b_spec, w_spec, b_spec,
        w_spec, b_spec, w_spec, b_spec,
        pl.BlockSpec((1, 1, DIM), lambda i: (0, 0, 0)),
        b_spec, b_spec,
    ]
    operands = [rows, xt2, Wd1, bd1, Wd2, bd2, Wc1, bc1, Wc2, bc2, sep, ln_g, ln_b]
    aliases = {}
    if prev is not None:
        in_specs = [pl.BlockSpec(memory_space=pl.ANY)] + in_specs
        operands = [prev] + operands
        aliases = {0: 0}
    return pl.pallas_call(
        _make_dense_body(bs, grid, base, prev is not None),
        grid=(grid,),
        in_specs=in_specs,
        out_specs=pl.BlockSpec(memory_space=pl.ANY),
        out_shape=jax.ShapeDtypeStruct((B_total, 8, DIM), jnp.float32),
        scratch_shapes=[
            pltpu.VMEM((bs, DIM), jnp.float32),
            pltpu.VMEM((2, bs, DIM), jnp.float32),
            pltpu.VMEM((2, bs, DIM), jnp.float32),
            pltpu.SemaphoreType.DMA((3, 2)),
        ],
        input_output_aliases=aliases,
        interpret=interpret,
    )(*operands)


def kernel(idx_genre, x_tempo, emb_table, Wd1, bd1, Wd2, bd2, Wc1, bc1,
           Wc2, bc2, sep_token, ln_g, ln_b):
    idx = idx_genre.astype(jnp.int32)
    B = idx.shape[0]
    H = B // 2
    rows0 = _sc_gather(emb_table, idx[:H])
    rows1 = _sc_gather(emb_table, idx[H:])
    args = (Wd1, bd1, Wd2, bd2, Wc1, bc1, Wc2, bc2, sep_token, ln_g, ln_b)
    out0 = _tc_dense_half(B, rows0, x_tempo[:H], *args, base=0)
    out8 = _tc_dense_half(B, rows1, x_tempo[H:], *args, base=H, prev=out0)
    return lax.slice(out8, (0, 0, 0), (B, 3, DIM))


# speedup vs baseline: 1.4883x; 1.4883x over previous
import functools
import math

import jax
import jax.numpy as jnp
from jax import lax
from jax.experimental import pallas as pl
from jax.experimental.pallas import tpu as pltpu
from jax.experimental.pallas import tpu_sc as plsc

DIM = 128
HALF = 64
RANGE_MAX = 250.0
LOG_THETA = math.log(10000.0)

_NC = 2
_NS = 16
_NW = _NC * _NS
_K = 128


def _sc_gather(table, idx4, half, Bh):
    b_per_w = Bh // _NW
    n_chunks = b_per_w // _K
    mesh = plsc.VectorSubcoreMesh(core_axis_name="c", subcore_axis_name="s")

    @functools.partial(
        pl.kernel,
        mesh=mesh,
        out_type=jax.ShapeDtypeStruct((Bh, DIM), jnp.float32),
        scratch_types=[
            pltpu.VMEM((n_chunks, _K), jnp.int32),
            pltpu.VMEM((b_per_w, DIM), jnp.float32),
            pltpu.SemaphoreType.DMA,
        ],
    )
    def gather_kernel(table_hbm, idx_hbm, out_hbm, idx_v, rows_v, sem):
        wid = lax.axis_index("s") * _NC + lax.axis_index("c")
        pltpu.sync_copy(idx_hbm.at[half, wid], idx_v)
        copies = [
            pltpu.async_copy(
                table_hbm.at[idx_v.at[j]], rows_v.at[pl.ds(j * _K, _K)], sem
            )
            for j in range(n_chunks)
        ]
        for c in copies:
            c.wait()
        pltpu.sync_copy(rows_v, out_hbm.at[pl.ds(wid * b_per_w, b_per_w)])

    return gather_kernel(table, idx4)


def _sin01(x):
    x2 = x * x
    return x * (1.0 + x2 * (-1.0 / 6.0 + x2 * (1.0 / 120.0 + x2 * (-1.0 / 5040.0))))


def _cos01(x):
    x2 = x * x
    return 1.0 + x2 * (-0.5 + x2 * (1.0 / 24.0 + x2 * (-1.0 / 720.0 + x2 * (1.0 / 40320.0))))


def _make_dense_body(bs, grid, base, aliased):
    def body(*refs):
        if aliased:
            refs = refs[1:]
        (rows_ref, xt_ref, wd1, bd1, wd2, bd2, wc1, bc1, wc2, bc2,
         sep_ref, g_ref, b_ref, out_ref, sep_buf, d_buf, c_buf, sems) = refs
        i = pl.program_id(0)
        slot = lax.rem(i, 2)
        g = g_ref[...].reshape(1, DIM)
        b = b_ref[...].reshape(1, DIM)

        def ln(x):
            m = jnp.mean(x, axis=-1, keepdims=True)
            c = x - m
            v = jnp.mean(c * c, axis=-1, keepdims=True)
            return c * lax.rsqrt(v + 1e-5) * g + b

        def plane_dma(buf, step, plane, sem):
            return pltpu.make_async_copy(
                buf, out_ref.at[pl.ds(base + step * bs, bs), plane, :], sem)

        def wait_step(step, s):
            plane_dma(sep_buf, step, 0, sems.at[0, s]).wait()
            plane_dma(d_buf.at[s], step, 1, sems.at[1, s]).wait()
            plane_dma(c_buf.at[s], step, 2, sems.at[2, s]).wait()

        @pl.when(i >= 2)
        def _drain_old():
            wait_step(i - 2, slot)

        @pl.when(i == 0)
        def _fill_sep():
            sep_n = ln(sep_ref[...].reshape(1, DIM))
            sep_buf[...] = jnp.broadcast_to(sep_n, (bs, DIM))

        rows = rows_ref[...]
        h = jnp.dot(rows, wd1[...], preferred_element_type=jnp.float32)
        h = h + bd1[...].reshape(1, DIM)
        h = h * jax.nn.sigmoid(h)
        emb_d = jnp.dot(h, wd2[...], preferred_element_type=jnp.float32)
        emb_d = emb_d + bd2[...].reshape(1, DIM)
        d_buf[slot] = ln(emb_d)

        xc = jnp.clip(xt_ref[...].reshape(1, bs), 0.0, RANGE_MAX) * (1.0 / RANGE_MAX)
        f = lax.broadcasted_iota(jnp.int32, (HALF, 1), 0).astype(jnp.float32) * (1.0 / HALF)
        inv = jnp.exp(-LOG_THETA * f)
        ang = inv * xc
        feat_t = jnp.concatenate([_sin01(ang), _cos01(ang)], axis=0)
        h2 = lax.dot_general(feat_t, wc1[...], (((0,), (0,)), ((), ())),
                             preferred_element_type=jnp.float32)
        h2 = h2 + bc1[...].reshape(1, DIM)
        h2 = h2 * jax.nn.sigmoid(h2)
        emb_c = jnp.dot(h2, wc2[...], preferred_element_type=jnp.float32)
        emb_c = emb_c + bc2[...].reshape(1, DIM)
        c_buf[slot] = ln(emb_c)

        plane_dma(sep_buf, i, 0, sems.at[0, slot]).start()
        plane_dma(d_buf.at[slot], i, 1, sems.at[1, slot]).start()
        plane_dma(c_buf.at[slot], i, 2, sems.at[2, slot]).start()

        @pl.when(i == grid - 1)
        def _drain_tail():
            wait_step(i - 1, 1 - slot)
            wait_step(i, slot)

    return body


def _tc_dense_half(B_total, rows, xt, Wd1, bd1, Wd2, bd2, Wc1, bc1, Wc2, bc2,
                   sep, ln_g, ln_b, base, prev=None, interpret=False):
    Bh = rows.shape[0]
    bs = 4096
    grid = Bh // bs
    base_blk = base // bs
    w_spec = pl.BlockSpec((DIM, DIM), lambda i: (0, 0))
    b_spec = pl.BlockSpec((DIM,), lambda i: (0,))
    in_specs = [
        pl.BlockSpec((bs, DIM), lambda i: (i, 0)),
        pl.BlockSpec((1, 1, bs), lambda i: (base_blk + i, 0, 0)),
        w_spec, b_spec, w_spec, b_spec,
        w_spec, b_spec, w_spec, b_spec,
        pl.BlockSpec((1, 1, DIM), lambda i: (0, 0, 0)),
        b_spec, b_spec,
    ]
    operands = [rows, xt, Wd1, bd1, Wd2, bd2, Wc1, bc1, Wc2, bc2, sep, ln_g, ln_b]
    aliases = {}
    if prev is not None:
        in_specs = [pl.BlockSpec(memory_space=pl.ANY)] + in_specs
        operands = [prev] + operands
        aliases = {0: 0}
    return pl.pallas_call(
        _make_dense_body(bs, grid, base, prev is not None),
        grid=(grid,),
        in_specs=in_specs,
        out_specs=pl.BlockSpec(memory_space=pl.ANY),
        out_shape=jax.ShapeDtypeStruct((B_total, 3, DIM), jnp.float32),
        scratch_shapes=[
            pltpu.VMEM((bs, DIM), jnp.float32),
            pltpu.VMEM((2, bs, DIM), jnp.float32),
            pltpu.VMEM((2, bs, DIM), jnp.float32),
            pltpu.SemaphoreType.DMA((3, 2)),
        ],
        input_output_aliases=aliases,
        interpret=interpret,
    )(*operands)


def kernel(idx_genre, x_tempo, emb_table, Wd1, bd1, Wd2, bd2, Wc1, bc1,
           Wc2, bc2, sep_token, ln_g, ln_b):
    idx = idx_genre.astype(jnp.int32)
    B = idx.shape[0]
    H = B // 2
    bs = 4096
    idx4 = idx.reshape(2, _NW, (H // _NW) // _K, _K)
    xt3 = x_tempo.reshape(B // bs, 1, bs)
    rows0 = _sc_gather(emb_table, idx4, 0, H)
    rows1 = _sc_gather(emb_table, idx4, 1, H)
    args = (Wd1, bd1, Wd2, bd2, Wc1, bc1, Wc2, bc2, sep_token, ln_g, ln_b)
    out0 = _tc_dense_half(B, rows0, xt3, *args, base=0)
    return _tc_dense_half(B, rows1, xt3, *args, base=H, prev=out0)
